# baseline (speedup 1.0000x reference)
import functools

import jax
import jax.numpy as jnp
from jax import lax
from jax.experimental import pallas as pl
from jax.experimental.pallas import tpu as pltpu
from jax.experimental.pallas import tpu_sc as plsc

N = 10000
E = 320000
D = 128
H = 128
NE = 4
NC = 10
NG = 128
L = 3

_IP = False


def _gin_mlp_body(eps_ref, h_ref, a_ref, w1_ref, b1_ref, w2_ref, b2_ref,
                  o_ref, *, relu_out, nparts):
    agg = a_ref[0]
    for p in range(1, nparts):
        agg = agg + a_ref[p]
    hpre = (1.0 + eps_ref[0]) * h_ref[...] + agg
    hid = jnp.maximum(
        jnp.dot(hpre, w1_ref[...], preferred_element_type=jnp.float32)
        + b1_ref[...], 0.0)
    out = jnp.dot(hid, w2_ref[...], preferred_element_type=jnp.float32) \
        + b2_ref[...]
    if relu_out:
        out = jnp.maximum(out, 0.0)
    o_ref[...] = out


def _gin_mlp(h, agg_parts, w1, b1, w2, b2, eps, relu_out):
    R = h.shape[0]
    BR = 1000
    P = agg_parts.shape[0]
    grid = (R // BR,)
    return pl.pallas_call(
        functools.partial(_gin_mlp_body, relu_out=relu_out, nparts=P),
        grid=grid,
        in_specs=[
            pl.BlockSpec(memory_space=pltpu.SMEM),
            pl.BlockSpec((BR, H), lambda i: (i, 0)),
            pl.BlockSpec((P, BR, H), lambda i: (0, i, 0)),
            pl.BlockSpec((H, H), lambda i: (0, 0)),
            pl.BlockSpec((1, H), lambda i: (0, 0)),
            pl.BlockSpec((H, H), lambda i: (0, 0)),
            pl.BlockSpec((1, H), lambda i: (0, 0)),
        ],
        out_specs=pl.BlockSpec((BR, H), lambda i: (i, 0)),
        out_shape=jax.ShapeDtypeStruct((R, H), jnp.float32),
        interpret=_IP,
    )(eps.reshape(1), h, agg_parts, w1, b1.reshape(1, H), w2,
      b2.reshape(1, H))


def _mask_body(z_ref, x_ref, nw1_ref, nb1_ref, nbd2_ref, nb2_ref,
               fw1_ref, fb1_ref, fw2_ref, fb2_ref,
               eA_ref, eB_ref, eb1_ref,
               nmask_o, fmask_o, mx_o, p1_o, p2_o):
    z = z_ref[...]
    x = x_ref[...]
    nh = jnp.maximum(
        jnp.dot(z, nw1_ref[...], preferred_element_type=jnp.float32)
        + nb1_ref[...], 0.0)
    nlog = jnp.dot(nh, nbd2_ref[...], preferred_element_type=jnp.float32) \
        + nb2_ref[...]
    nmask = jax.nn.sigmoid(nlog * 10.0)
    nmask_o[...] = nmask
    fh = jnp.maximum(
        jnp.dot(z, fw1_ref[...], preferred_element_type=jnp.float32)
        + fb1_ref[...], 0.0)
    for e in range(NE):
        flog = jnp.dot(fh[:, e * H:(e + 1) * H], fw2_ref[e],
                       preferred_element_type=jnp.float32) + fb2_ref[e]
        fmask = jax.nn.sigmoid(flog * 10.0)
        fmask_o[e] = fmask
        mx_o[e] = x * nmask[:, e:e + 1] * fmask
    p1_o[...] = jnp.dot(z, eA_ref[...], preferred_element_type=jnp.float32) \
        + eb1_ref[...]
    p2_o[...] = jnp.dot(z, eB_ref[...], preferred_element_type=jnp.float32)


def _mask_stage(Z, x, nmW1, nmb1, nmW2, nmb2, fmW1, fmb1, fmW2, fmb2,
                emW1, emb1):
    BR = 1000
    grid = (N // BR,)
    HS = NE * H
    nw1 = nmW1.transpose(1, 0, 2).reshape(H, HS)
    nb1 = nmb1.reshape(1, HS)
    nbd2 = jnp.zeros((HS, NE), jnp.float32)
    for e in range(NE):
        nbd2 = nbd2.at[e * H:(e + 1) * H, e].set(nmW2[e, :, 0])
    nb2 = nmb2.reshape(1, NE)
    fw1 = fmW1.transpose(1, 0, 2).reshape(H, HS)
    fb1 = fmb1.reshape(1, HS)
    eA = emW1[:, :H, :].transpose(1, 0, 2).reshape(H, HS)
    eB = emW1[:, H:, :].transpose(1, 0, 2).reshape(H, HS)
    eb1 = emb1.reshape(1, HS)
    outs = pl.pallas_call(
        _mask_body,
        grid=grid,
        in_specs=[
            pl.BlockSpec((BR, H), lambda i: (i, 0)),
            pl.BlockSpec((BR, H), lambda i: (i, 0)),
            pl.BlockSpec((H, HS), lambda i: (0, 0)),
            pl.BlockSpec((1, HS), lambda i: (0, 0)),
            pl.BlockSpec((HS, NE), lambda i: (0, 0)),
            pl.BlockSpec((1, NE), lambda i: (0, 0)),
            pl.BlockSpec((H, HS), lambda i: (0, 0)),
            pl.BlockSpec((1, HS), lambda i: (0, 0)),
            pl.BlockSpec((NE, H, H), lambda i: (0, 0, 0)),
            pl.BlockSpec((NE, 1, H), lambda i: (0, 0, 0)),
            pl.BlockSpec((H, HS), lambda i: (0, 0)),
            pl.BlockSpec((H, HS), lambda i: (0, 0)),
            pl.BlockSpec((1, HS), lambda i: (0, 0)),
        ],
        out_specs=[
            pl.BlockSpec((BR, NE), lambda i: (i, 0)),
            pl.BlockSpec((NE, BR, H), lambda i: (0, i, 0)),
            pl.BlockSpec((NE, BR, H), lambda i: (0, i, 0)),
            pl.BlockSpec((BR, HS), lambda i: (i, 0)),
            pl.BlockSpec((BR, HS), lambda i: (i, 0)),
        ],
        out_shape=[
            jax.ShapeDtypeStruct((N, NE), jnp.float32),
            jax.ShapeDtypeStruct((NE, N, H), jnp.float32),
            jax.ShapeDtypeStruct((NE, N, H), jnp.float32),
            jax.ShapeDtypeStruct((N, HS), jnp.float32),
            jax.ShapeDtypeStruct((N, HS), jnp.float32),
        ],
        interpret=_IP,
    )(Z, x, nw1, nb1, nbd2, nb2, fw1, fb1, fmW2, fmb2.reshape(NE, 1, H),
      eA, eB, eb1)
    return outs


def _edge_mlp_body(p1_ref, p2_ref, bd2_ref, b2_ref, o_ref):
    hid = jnp.maximum(p1_ref[...] + p2_ref[...], 0.0)
    elog = jnp.dot(hid, bd2_ref[...], preferred_element_type=jnp.float32) \
        + b2_ref[...]
    o_ref[...] = jax.nn.sigmoid(elog * 10.0)


def _edge_mlp(P1s, P2d, emW2, emb2):
    BE = 2000
    HS = NE * H
    bd2 = jnp.zeros((HS, NE), jnp.float32)
    for e in range(NE):
        bd2 = bd2.at[e * H:(e + 1) * H, e].set(emW2[e, :, 0])
    return pl.pallas_call(
        _edge_mlp_body,
        grid=(E // BE,),
        in_specs=[
            pl.BlockSpec((BE, HS), lambda i: (i, 0)),
            pl.BlockSpec((BE, HS), lambda i: (i, 0)),
            pl.BlockSpec((HS, NE), lambda i: (0, 0)),
            pl.BlockSpec((1, NE), lambda i: (0, 0)),
        ],
        out_specs=pl.BlockSpec((BE, NE), lambda i: (i, 0)),
        out_shape=jax.ShapeDtypeStruct((E, NE), jnp.float32),
        interpret=_IP,
    )(P1s, P2d, bd2, emb2.reshape(1, NE))


def _pool_body(h_ref, b_ref, o_ref, acc, cnt, *, G, C, nsteps):
    i = pl.program_id(0)

    @pl.when(i == 0)
    def _init():
        acc[...] = jnp.zeros_like(acc)
        cnt[...] = jnp.zeros_like(cnt)

    b = b_ref[0, 0, :]
    gid = lax.broadcasted_iota(jnp.int32, (NG, b.shape[0]), 0)
    onehot = (b[None, :] == gid).astype(jnp.float32)
    cnt[...] += jnp.sum(onehot, axis=1, keepdims=True)
    for g in range(G):
        acc[g] += jnp.dot(onehot, h_ref[g],
                          preferred_element_type=jnp.float32)

    @pl.when(i == nsteps - 1)
    def _fini():
        c = jnp.maximum(cnt[...], 1.0)
        for g in range(G):
            o_ref[g] = acc[g] / c


def _pool(h, batch3):
    G, _, C = h.shape
    BR = 1000
    nsteps = N // BR
    return pl.pallas_call(
        functools.partial(_pool_body, G=G, C=C, nsteps=nsteps),
        grid=(nsteps,),
        in_specs=[
            pl.BlockSpec((G, BR, C), lambda i: (0, i, 0)),
            pl.BlockSpec((1, 1, BR), lambda i: (i, 0, 0)),
        ],
        out_specs=pl.BlockSpec((G, NG, C), lambda i: (0, 0, 0)),
        out_shape=jax.ShapeDtypeStruct((G, NG, C), jnp.float32),
        scratch_shapes=[pltpu.VMEM((G, NG, C), jnp.float32),
                        pltpu.VMEM((NG, 1), jnp.float32)],
        interpret=_IP,
    )(h, batch3)


def _cls_body(h_ref, w_ref, b_ref, o_ref):
    for e in range(NE):
        o_ref[e] = jnp.dot(h_ref[e], w_ref[e],
                           preferred_element_type=jnp.float32) + b_ref[e]


def _cls(h_st, clsW, clsb):
    return pl.pallas_call(
        _cls_body,
        grid=(1,),
        in_specs=[
            pl.BlockSpec((NE, NG, H), lambda i: (0, 0, 0)),
            pl.BlockSpec((NE, H, NC), lambda i: (0, 0, 0)),
            pl.BlockSpec((NE, 1, NC), lambda i: (0, 0, 0)),
        ],
        out_specs=pl.BlockSpec((NE, NG, NC), lambda i: (0, 0, 0)),
        out_shape=jax.ShapeDtypeStruct((NE, NG, NC), jnp.float32),
        interpret=_IP,
    )(h_st, clsW, clsb.reshape(NE, 1, NC))


def _agg_jax(h, src, dst, w):
    G = h.shape[0] // N
    hh = h.reshape(G, N, H)
    def one(he, we):
        msg = he[src] * we[:, None]
        return jax.ops.segment_sum(msg, dst, num_segments=N)
    if w is None:
        w = jnp.ones((G, E), jnp.float32)
    agg = jax.vmap(one)(hh, w)
    return agg.reshape(1, G * N, H)


def kernel(x, edge_index, batch, enc1_W1, enc1_b1, enc1_W2, enc1_b2,
           enc1_eps, enc2_W1, enc2_b1, enc2_W2, enc2_b2, enc2_eps,
           nmW1, nmb1, nmW2, nmb2, emW1, emb1, emW2, emb2,
           fmW1, fmb1, fmW2, fmb2, clsW, clsb):
    src = edge_index[0]
    dst = edge_index[1]
    batch3 = batch.reshape(N // 1000, 1, 1000)

    h = x
    for l in range(L):
        agg = _agg_jax(h, src, dst, None)
        h = _gin_mlp(h, agg, enc1_W1[l], enc1_b1[l], enc1_W2[l],
                     enc1_b2[l], enc1_eps[l], relu_out=(l < L - 1))
    Z = h

    nmask, fmaskT, mx, P1, P2 = _mask_stage(
        Z, x, nmW1, nmb1, nmW2, nmb2, fmW1, fmb1, fmW2, fmb2, emW1, emb1)

    P1s = P1[src]
    P2d = P2[dst]
    emask = _edge_mlp(P1s, P2d, emW2, emb2)
    emaskT = emask.T

    h = mx.reshape(NE * N, H)
    for l in range(L):
        agg = _agg_jax(h, src, dst, emaskT)
        h = _gin_mlp(h, agg, enc2_W1[l], enc2_b1[l], enc2_W2[l],
                     enc2_b2[l], enc2_eps[l], relu_out=(l < L - 1))
    mZ = h.reshape(NE, N, H)

    h_orig = _pool(Z.reshape(1, N, H), batch3)[0]
    h_st = _pool(mZ, batch3)
    logits = _cls(h_st, clsW, clsb)

    expert_logits = logits.transpose(1, 0, 2)
    h_stable = h_st.transpose(1, 0, 2)
    node_masks = nmask[:, :, None]
    edge_masks = emask[:, :, None]
    feat_masks = fmaskT.transpose(1, 0, 2)
    return (expert_logits, h_stable, h_orig, node_masks, edge_masks,
            feat_masks)


# speedup vs baseline: 15.5129x; 15.5129x over previous
import functools

import jax
import jax.numpy as jnp
from jax import lax
from jax.experimental import pallas as pl
from jax.experimental.pallas import tpu as pltpu
from jax.experimental.pallas import tpu_sc as plsc

N = 10000
E = 320000
D = 128
H = 128
NE = 4
NC = 10
NG = 128
L = 3

_IP = False


def _gin_mlp_body(eps_ref, h_ref, a_ref, w1_ref, b1_ref, w2_ref, b2_ref,
                  o_ref, *, relu_out, nparts):
    agg = a_ref[0]
    for p in range(1, nparts):
        agg = agg + a_ref[p]
    hpre = (1.0 + eps_ref[0]) * h_ref[...] + agg
    hid = jnp.maximum(
        jnp.dot(hpre, w1_ref[...], preferred_element_type=jnp.float32)
        + b1_ref[...], 0.0)
    out = jnp.dot(hid, w2_ref[...], preferred_element_type=jnp.float32) \
        + b2_ref[...]
    if relu_out:
        out = jnp.maximum(out, 0.0)
    o_ref[...] = out


def _gin_mlp(h, agg_parts, w1, b1, w2, b2, eps, relu_out):
    R = h.shape[0]
    BR = 1000
    P = agg_parts.shape[0]
    grid = (R // BR,)
    return pl.pallas_call(
        functools.partial(_gin_mlp_body, relu_out=relu_out, nparts=P),
        grid=grid,
        in_specs=[
            pl.BlockSpec(memory_space=pltpu.SMEM),
            pl.BlockSpec((BR, H), lambda i: (i, 0)),
            pl.BlockSpec((P, BR, H), lambda i: (0, i, 0)),
            pl.BlockSpec((H, H), lambda i: (0, 0)),
            pl.BlockSpec((1, H), lambda i: (0, 0)),
            pl.BlockSpec((H, H), lambda i: (0, 0)),
            pl.BlockSpec((1, H), lambda i: (0, 0)),
        ],
        out_specs=pl.BlockSpec((BR, H), lambda i: (i, 0)),
        out_shape=jax.ShapeDtypeStruct((R, H), jnp.float32),
        interpret=_IP,
    )(eps.reshape(1), h, agg_parts, w1, b1.reshape(1, H), w2,
      b2.reshape(1, H))


def _mask_body(z_ref, x_ref, nw1_ref, nb1_ref, nbd2_ref, nb2_ref,
               fw1_ref, fb1_ref, fw2_ref, fb2_ref,
               nmask_o, fmask_o, mx_o):
    z = z_ref[...]
    x = x_ref[...]
    nh = jnp.maximum(
        jnp.dot(z, nw1_ref[...], preferred_element_type=jnp.float32)
        + nb1_ref[...], 0.0)
    nlog = jnp.dot(nh, nbd2_ref[...], preferred_element_type=jnp.float32) \
        + nb2_ref[...]
    nmask = jax.nn.sigmoid(nlog * 10.0)
    nmask_o[...] = nmask
    fh = jnp.maximum(
        jnp.dot(z, fw1_ref[...], preferred_element_type=jnp.float32)
        + fb1_ref[...], 0.0)
    for e in range(NE):
        flog = jnp.dot(fh[:, e * H:(e + 1) * H], fw2_ref[e],
                       preferred_element_type=jnp.float32) + fb2_ref[e]
        fmask = jax.nn.sigmoid(flog * 10.0)
        fmask_o[e] = fmask
        mx_o[e] = x * nmask[:, e:e + 1] * fmask


def _mask_stage(Z, x, nmW1, nmb1, nmW2, nmb2, fmW1, fmb1, fmW2, fmb2):
    BR = 1000
    grid = (N // BR,)
    HS = NE * H
    nw1 = nmW1.transpose(1, 0, 2).reshape(H, HS)
    nb1 = nmb1.reshape(1, HS)
    nbd2 = jnp.zeros((HS, NE), jnp.float32)
    for e in range(NE):
        nbd2 = nbd2.at[e * H:(e + 1) * H, e].set(nmW2[e, :, 0])
    nb2 = nmb2.reshape(1, NE)
    fw1 = fmW1.transpose(1, 0, 2).reshape(H, HS)
    fb1 = fmb1.reshape(1, HS)
    outs = pl.pallas_call(
        _mask_body,
        grid=grid,
        in_specs=[
            pl.BlockSpec((BR, H), lambda i: (i, 0)),
            pl.BlockSpec((BR, H), lambda i: (i, 0)),
            pl.BlockSpec((H, HS), lambda i: (0, 0)),
            pl.BlockSpec((1, HS), lambda i: (0, 0)),
            pl.BlockSpec((HS, NE), lambda i: (0, 0)),
            pl.BlockSpec((1, NE), lambda i: (0, 0)),
            pl.BlockSpec((H, HS), lambda i: (0, 0)),
            pl.BlockSpec((1, HS), lambda i: (0, 0)),
            pl.BlockSpec((NE, H, H), lambda i: (0, 0, 0)),
            pl.BlockSpec((NE, 1, H), lambda i: (0, 0, 0)),
        ],
        out_specs=[
            pl.BlockSpec((BR, NE), lambda i: (i, 0)),
            pl.BlockSpec((NE, BR, H), lambda i: (0, i, 0)),
            pl.BlockSpec((NE, BR, H), lambda i: (0, i, 0)),
        ],
        out_shape=[
            jax.ShapeDtypeStruct((N, NE), jnp.float32),
            jax.ShapeDtypeStruct((NE, N, H), jnp.float32),
            jax.ShapeDtypeStruct((NE, N, H), jnp.float32),
        ],
        interpret=_IP,
    )(Z, x, nw1, nb1, nbd2, nb2, fw1, fb1, fmW2, fmb2.reshape(NE, 1, H))
    return outs


def _edge_mlp_body(zs_ref, zd_ref, eA_ref, eB_ref, eb1_ref, bd2_ref, b2_ref,
                   o_ref):
    hid = jnp.maximum(
        jnp.dot(zs_ref[...], eA_ref[...], preferred_element_type=jnp.float32)
        + jnp.dot(zd_ref[...], eB_ref[...],
                  preferred_element_type=jnp.float32)
        + eb1_ref[...], 0.0)
    elog = jnp.dot(hid, bd2_ref[...], preferred_element_type=jnp.float32) \
        + b2_ref[...]
    o_ref[...] = jax.nn.sigmoid(elog * 10.0)


def _edge_mlp(Zs, Zd, emW1, emb1, emW2, emb2):
    BE = 2000
    HS = NE * H
    eA = emW1[:, :H, :].transpose(1, 0, 2).reshape(H, HS)
    eB = emW1[:, H:, :].transpose(1, 0, 2).reshape(H, HS)
    bd2 = jnp.zeros((HS, NE), jnp.float32)
    for e in range(NE):
        bd2 = bd2.at[e * H:(e + 1) * H, e].set(emW2[e, :, 0])
    return pl.pallas_call(
        _edge_mlp_body,
        grid=(E // BE,),
        in_specs=[
            pl.BlockSpec((BE, H), lambda i: (i, 0)),
            pl.BlockSpec((BE, H), lambda i: (i, 0)),
            pl.BlockSpec((H, HS), lambda i: (0, 0)),
            pl.BlockSpec((H, HS), lambda i: (0, 0)),
            pl.BlockSpec((1, HS), lambda i: (0, 0)),
            pl.BlockSpec((HS, NE), lambda i: (0, 0)),
            pl.BlockSpec((1, NE), lambda i: (0, 0)),
        ],
        out_specs=pl.BlockSpec((BE, NE), lambda i: (i, 0)),
        out_shape=jax.ShapeDtypeStruct((E, NE), jnp.float32),
        interpret=_IP,
    )(Zs, Zd, eA, eB, emb1.reshape(1, HS), bd2, emb2.reshape(1, NE))


def _pool_body(h_ref, b_ref, o_ref, acc, cnt, *, G, C, nsteps):
    i = pl.program_id(0)

    @pl.when(i == 0)
    def _init():
        acc[...] = jnp.zeros_like(acc)
        cnt[...] = jnp.zeros_like(cnt)

    b = b_ref[0, 0, :]
    gid = lax.broadcasted_iota(jnp.int32, (NG, b.shape[0]), 0)
    onehot = (b[None, :] == gid).astype(jnp.float32)
    cnt[...] += jnp.sum(onehot, axis=1, keepdims=True)
    for g in range(G):
        acc[g] += jnp.dot(onehot, h_ref[g],
                          preferred_element_type=jnp.float32)

    @pl.when(i == nsteps - 1)
    def _fini():
        c = jnp.maximum(cnt[...], 1.0)
        for g in range(G):
            o_ref[g] = acc[g] / c


def _pool(h, batch3):
    G, _, C = h.shape
    BR = 1000
    nsteps = N // BR
    return pl.pallas_call(
        functools.partial(_pool_body, G=G, C=C, nsteps=nsteps),
        grid=(nsteps,),
        in_specs=[
            pl.BlockSpec((G, BR, C), lambda i: (0, i, 0)),
            pl.BlockSpec((1, 1, BR), lambda i: (i, 0, 0)),
        ],
        out_specs=pl.BlockSpec((G, NG, C), lambda i: (0, 0, 0)),
        out_shape=jax.ShapeDtypeStruct((G, NG, C), jnp.float32),
        scratch_shapes=[pltpu.VMEM((G, NG, C), jnp.float32),
                        pltpu.VMEM((NG, 1), jnp.float32)],
        interpret=_IP,
    )(h, batch3)


def _cls_body(h_ref, w_ref, b_ref, o_ref):
    for e in range(NE):
        o_ref[e] = jnp.dot(h_ref[e], w_ref[e],
                           preferred_element_type=jnp.float32) + b_ref[e]


def _cls(h_st, clsW, clsb):
    return pl.pallas_call(
        _cls_body,
        grid=(1,),
        in_specs=[
            pl.BlockSpec((NE, NG, H), lambda i: (0, 0, 0)),
            pl.BlockSpec((NE, H, NC), lambda i: (0, 0, 0)),
            pl.BlockSpec((NE, 1, NC), lambda i: (0, 0, 0)),
        ],
        out_specs=pl.BlockSpec((NE, NG, NC), lambda i: (0, 0, 0)),
        out_shape=jax.ShapeDtypeStruct((NE, NG, NC), jnp.float32),
        interpret=_IP,
    )(h_st, clsW, clsb.reshape(NE, 1, NC))


_NCORES = 2
_NSUB = 16
_NW = _NCORES * _NSUB
_EPT = E // _NW
_KE = 80
_NCH = _EPT // _KE
_RPT = N // _NSUB
_ZR = 25
_NCH16 = _KE // 16


def _sc_mesh():
    return plsc.VectorSubcoreMesh(core_axis_name="c", subcore_axis_name="s",
                                  num_cores=_NCORES, num_subcores=_NSUB)


def _sc_agg(h, src1, dst1, w, nexp, weighted):
    R = nexp * N
    scratch = [
        pltpu.VMEM((_KE,), jnp.int32),
        pltpu.VMEM((_KE,), jnp.int32),
        pltpu.VMEM((_KE,), jnp.float32),
        pltpu.VMEM((_KE, H), jnp.float32),
        pltpu.VMEM((_ZR, H), jnp.float32),
        pltpu.VMEM_SHARED((N, H), jnp.float32),
        pltpu.SemaphoreType.DMA,
    ]

    def body(h_hbm, src_hbm, dst_hbm, w_hbm, out_hbm,
             sidx, didx, wv, rows, zbuf, acc, sem):
        c = lax.axis_index("c")
        s = lax.axis_index("s")
        wid = s * _NCORES + c
        eoff = wid * _EPT
        @pl.loop(0, _ZR)
        def _zb(i):
            for j in range(H // 16):
                zbuf[i, pl.ds(16 * j, 16)] = jnp.zeros((16,), jnp.float32)

        for e in range(nexp):
            for z in range(_RPT // _ZR):
                pltpu.sync_copy(zbuf, acc.at[pl.ds(s * _RPT + z * _ZR, _ZR), :])
            plsc.subcore_barrier()

            @pl.loop(0, _NCH)
            def _chunk(i):
                pltpu.sync_copy(src_hbm.at[pl.ds(eoff + i * _KE, _KE)], sidx)
                if e > 0:
                    for j in range(_KE // 16):
                        sidx[pl.ds(16 * j, 16)] = (
                            sidx[pl.ds(16 * j, 16)] + e * N)
                pltpu.async_copy(h_hbm.at[sidx], rows, sem).wait()
                if weighted:
                    pltpu.sync_copy(
                        w_hbm.at[e, pl.ds(eoff + i * _KE, _KE)], wv)
                    for k16 in range(_NCH16):
                        wvec = wv[pl.ds(k16 * 16, 16)]
                        for k2 in range(16):
                            wb = jnp.full((16,), wvec[k2], jnp.float32)
                            r = k16 * 16 + k2
                            for j in range(H // 16):
                                rows[r, pl.ds(16 * j, 16)] = (
                                    rows[r, pl.ds(16 * j, 16)] * wb)
                pltpu.sync_copy(dst_hbm.at[pl.ds(eoff + i * _KE, _KE)], didx)
                pltpu.sync_copy(rows, acc.at[didx], add=True)

            plsc.subcore_barrier()
            pltpu.sync_copy(
                acc.at[pl.ds(s * _RPT, _RPT), :],
                out_hbm.at[c, pl.ds(e * N + s * _RPT, _RPT), :])
            plsc.subcore_barrier()

    if w is None:
        w = jnp.zeros((nexp, E), jnp.float32)
    return pl.kernel(
        body,
        out_type=jax.ShapeDtypeStruct((_NCORES, R, H), jnp.float32),
        mesh=_sc_mesh(),
        scratch_types=scratch,
        compiler_params=pltpu.CompilerParams(use_tc_tiling_on_sc=False),
    )(h, src1, dst1, w)


def _sc_edge_gather(Z, src1, dst1):
    scratch = [
        pltpu.VMEM((_KE,), jnp.int32),
        pltpu.VMEM((_KE, H), jnp.float32),
        pltpu.SemaphoreType.DMA,
    ]

    def body(z_hbm, src_hbm, dst_hbm, out_hbm, idx, rows, sem):
        c = lax.axis_index("c")
        s = lax.axis_index("s")
        wid = s * _NCORES + c
        eoff = wid * _EPT
        for which in range(2):
            ih = src_hbm if which == 0 else dst_hbm

            @pl.loop(0, _NCH)
            def _chunk(i):
                pltpu.sync_copy(ih.at[pl.ds(eoff + i * _KE, _KE)], idx)
                pltpu.async_copy(z_hbm.at[idx], rows, sem).wait()
                pltpu.sync_copy(
                    rows,
                    out_hbm.at[which, pl.ds(eoff + i * _KE, _KE), :])

    return pl.kernel(
        body,
        out_type=jax.ShapeDtypeStruct((2, E, H), jnp.float32),
        mesh=_sc_mesh(),
        scratch_types=scratch,
        compiler_params=pltpu.CompilerParams(use_tc_tiling_on_sc=False),
    )(Z, src1, dst1)


def _agg_jax(h, src, dst, w, nexp):
    hh = h.reshape(nexp, N, H)
    if w is None:
        w = jnp.ones((nexp, E), jnp.float32)
    def one(he, we):
        msg = he[src] * we[:, None]
        return jax.ops.segment_sum(msg, dst, num_segments=N)
    agg = jax.vmap(one)(hh, w)
    return agg.reshape(1, nexp * N, H)


def kernel(x, edge_index, batch, enc1_W1, enc1_b1, enc1_W2, enc1_b2,
           enc1_eps, enc2_W1, enc2_b1, enc2_W2, enc2_b2, enc2_eps,
           nmW1, nmb1, nmW2, nmb2, emW1, emb1, emW2, emb2,
           fmW1, fmb1, fmW2, fmb2, clsW, clsb):
    src1 = edge_index[0]
    dst1 = edge_index[1]
    batch3 = batch.reshape(N // 1000, 1, 1000)

    h = x
    for l in range(L):
        agg = _sc_agg(h, src1, dst1, None, 1, False)
        h = _gin_mlp(h, agg, enc1_W1[l], enc1_b1[l], enc1_W2[l],
                     enc1_b2[l], enc1_eps[l], relu_out=(l < L - 1))
    Z = h

    nmask, fmaskT, mx = _mask_stage(
        Z, x, nmW1, nmb1, nmW2, nmb2, fmW1, fmb1, fmW2, fmb2)

    ZsZd = _sc_edge_gather(Z, src1, dst1)
    emask = _edge_mlp(ZsZd[0], ZsZd[1], emW1, emb1, emW2, emb2)
    emaskT = emask.T

    h = mx.reshape(NE * N, H)
    for l in range(L):
        agg = _sc_agg(h, src1, dst1, emaskT, NE, True)
        h = _gin_mlp(h, agg, enc2_W1[l], enc2_b1[l], enc2_W2[l],
                     enc2_b2[l], enc2_eps[l], relu_out=(l < L - 1))
    mZ = h.reshape(NE, N, H)

    h_orig = _pool(Z.reshape(1, N, H), batch3)[0]
    h_st = _pool(mZ, batch3)
    logits = _cls(h_st, clsW, clsb)

    expert_logits = logits.transpose(1, 0, 2)
    h_stable = h_st.transpose(1, 0, 2)
    node_masks = nmask[:, :, None]
    edge_masks = emask[:, :, None]
    feat_masks = fmaskT.transpose(1, 0, 2)
    return (expert_logits, h_stable, h_orig, node_masks, edge_masks,
            feat_masks)


# speedup vs baseline: 25.5950x; 1.6499x over previous
import functools

import jax
import jax.numpy as jnp
from jax import lax
from jax.experimental import pallas as pl
from jax.experimental.pallas import tpu as pltpu
from jax.experimental.pallas import tpu_sc as plsc

N = 10000
E = 320000
D = 128
H = 128
NE = 4
NC = 10
NG = 128
L = 3

_IP = False


def _gin_mlp_body(eps_ref, h_ref, a_ref, w1_ref, b1_ref, w2_ref, b2_ref,
                  o_ref, *, relu_out, nparts):
    agg = a_ref[0]
    for p in range(1, nparts):
        agg = agg + a_ref[p]
    hpre = (1.0 + eps_ref[0]) * h_ref[...] + agg
    hid = jnp.maximum(
        jnp.dot(hpre, w1_ref[...], preferred_element_type=jnp.float32)
        + b1_ref[...], 0.0)
    out = jnp.dot(hid, w2_ref[...], preferred_element_type=jnp.float32) \
        + b2_ref[...]
    if relu_out:
        out = jnp.maximum(out, 0.0)
    o_ref[...] = out


def _gin_mlp(h, agg_parts, w1, b1, w2, b2, eps, relu_out):
    R = h.shape[0]
    BR = 1000
    P = agg_parts.shape[0]
    grid = (R // BR,)
    return pl.pallas_call(
        functools.partial(_gin_mlp_body, relu_out=relu_out, nparts=P),
        grid=grid,
        in_specs=[
            pl.BlockSpec(memory_space=pltpu.SMEM),
            pl.BlockSpec((BR, H), lambda i: (i, 0)),
            pl.BlockSpec((P, BR, H), lambda i: (0, i, 0)),
            pl.BlockSpec((H, H), lambda i: (0, 0)),
            pl.BlockSpec((1, H), lambda i: (0, 0)),
            pl.BlockSpec((H, H), lambda i: (0, 0)),
            pl.BlockSpec((1, H), lambda i: (0, 0)),
        ],
        out_specs=pl.BlockSpec((BR, H), lambda i: (i, 0)),
        out_shape=jax.ShapeDtypeStruct((R, H), jnp.float32),
        interpret=_IP,
    )(eps.reshape(1), h, agg_parts, w1, b1.reshape(1, H), w2,
      b2.reshape(1, H))


def _mask_body(z_ref, x_ref, nw1_ref, nb1_ref, nbd2_ref, nb2_ref,
               fw1_ref, fb1_ref, fw2_ref, fb2_ref,
               nmask_o, fmask_o, mx_o):
    z = z_ref[...]
    x = x_ref[...]
    nh = jnp.maximum(
        jnp.dot(z, nw1_ref[...], preferred_element_type=jnp.float32)
        + nb1_ref[...], 0.0)
    nlog = jnp.dot(nh, nbd2_ref[...], preferred_element_type=jnp.float32) \
        + nb2_ref[...]
    nmask = jax.nn.sigmoid(nlog * 10.0)
    nmask_o[...] = nmask
    fh = jnp.maximum(
        jnp.dot(z, fw1_ref[...], preferred_element_type=jnp.float32)
        + fb1_ref[...], 0.0)
    for e in range(NE):
        flog = jnp.dot(fh[:, e * H:(e + 1) * H], fw2_ref[e],
                       preferred_element_type=jnp.float32) + fb2_ref[e]
        fmask = jax.nn.sigmoid(flog * 10.0)
        fmask_o[e] = fmask
        mx_o[e] = x * nmask[:, e:e + 1] * fmask


def _mask_stage(Z, x, nmW1, nmb1, nmW2, nmb2, fmW1, fmb1, fmW2, fmb2):
    BR = 1000
    grid = (N // BR,)
    HS = NE * H
    nw1 = nmW1.transpose(1, 0, 2).reshape(H, HS)
    nb1 = nmb1.reshape(1, HS)
    nbd2 = jnp.zeros((HS, NE), jnp.float32)
    for e in range(NE):
        nbd2 = nbd2.at[e * H:(e + 1) * H, e].set(nmW2[e, :, 0])
    nb2 = nmb2.reshape(1, NE)
    fw1 = fmW1.transpose(1, 0, 2).reshape(H, HS)
    fb1 = fmb1.reshape(1, HS)
    outs = pl.pallas_call(
        _mask_body,
        grid=grid,
        in_specs=[
            pl.BlockSpec((BR, H), lambda i: (i, 0)),
            pl.BlockSpec((BR, H), lambda i: (i, 0)),
            pl.BlockSpec((H, HS), lambda i: (0, 0)),
            pl.BlockSpec((1, HS), lambda i: (0, 0)),
            pl.BlockSpec((HS, NE), lambda i: (0, 0)),
            pl.BlockSpec((1, NE), lambda i: (0, 0)),
            pl.BlockSpec((H, HS), lambda i: (0, 0)),
            pl.BlockSpec((1, HS), lambda i: (0, 0)),
            pl.BlockSpec((NE, H, H), lambda i: (0, 0, 0)),
            pl.BlockSpec((NE, 1, H), lambda i: (0, 0, 0)),
        ],
        out_specs=[
            pl.BlockSpec((BR, NE), lambda i: (i, 0)),
            pl.BlockSpec((NE, BR, H), lambda i: (0, i, 0)),
            pl.BlockSpec((NE, BR, H), lambda i: (0, i, 0)),
        ],
        out_shape=[
            jax.ShapeDtypeStruct((N, NE), jnp.float32),
            jax.ShapeDtypeStruct((NE, N, H), jnp.float32),
            jax.ShapeDtypeStruct((NE, N, H), jnp.float32),
        ],
        interpret=_IP,
    )(Z, x, nw1, nb1, nbd2, nb2, fw1, fb1, fmW2, fmb2.reshape(NE, 1, H))
    return outs


def _edge_mlp_body(zs_ref, zd_ref, eA_ref, eB_ref, eb1_ref, bd2_ref, b2_ref,
                   o_ref):
    hid = jnp.maximum(
        jnp.dot(zs_ref[...], eA_ref[...], preferred_element_type=jnp.float32)
        + jnp.dot(zd_ref[...], eB_ref[...],
                  preferred_element_type=jnp.float32)
        + eb1_ref[...], 0.0)
    elog = jnp.dot(hid, bd2_ref[...], preferred_element_type=jnp.float32) \
        + b2_ref[...]
    o_ref[...] = jax.nn.sigmoid(elog * 10.0)


def _edge_mlp(Zs, Zd, emW1, emb1, emW2, emb2):
    BE = 2000
    HS = NE * H
    eA = emW1[:, :H, :].transpose(1, 0, 2).reshape(H, HS)
    eB = emW1[:, H:, :].transpose(1, 0, 2).reshape(H, HS)
    bd2 = jnp.zeros((HS, NE), jnp.float32)
    for e in range(NE):
        bd2 = bd2.at[e * H:(e + 1) * H, e].set(emW2[e, :, 0])
    return pl.pallas_call(
        _edge_mlp_body,
        grid=(E // BE,),
        in_specs=[
            pl.BlockSpec((BE, H), lambda i: (i, 0)),
            pl.BlockSpec((BE, H), lambda i: (i, 0)),
            pl.BlockSpec((H, HS), lambda i: (0, 0)),
            pl.BlockSpec((H, HS), lambda i: (0, 0)),
            pl.BlockSpec((1, HS), lambda i: (0, 0)),
            pl.BlockSpec((HS, NE), lambda i: (0, 0)),
            pl.BlockSpec((1, NE), lambda i: (0, 0)),
        ],
        out_specs=pl.BlockSpec((BE, NE), lambda i: (i, 0)),
        out_shape=jax.ShapeDtypeStruct((E, NE), jnp.float32),
        interpret=_IP,
    )(Zs, Zd, eA, eB, emb1.reshape(1, HS), bd2, emb2.reshape(1, NE))


def _pool_body(h_ref, b_ref, o_ref, acc, cnt, *, G, C, nsteps):
    i = pl.program_id(0)

    @pl.when(i == 0)
    def _init():
        acc[...] = jnp.zeros_like(acc)
        cnt[...] = jnp.zeros_like(cnt)

    b = b_ref[0, 0, :]
    gid = lax.broadcasted_iota(jnp.int32, (NG, b.shape[0]), 0)
    onehot = (b[None, :] == gid).astype(jnp.float32)
    cnt[...] += jnp.sum(onehot, axis=1, keepdims=True)
    for g in range(G):
        acc[g] += jnp.dot(onehot, h_ref[g],
                          preferred_element_type=jnp.float32)

    @pl.when(i == nsteps - 1)
    def _fini():
        c = jnp.maximum(cnt[...], 1.0)
        for g in range(G):
            o_ref[g] = acc[g] / c


def _pool(h, batch3):
    G, _, C = h.shape
    BR = 1000
    nsteps = N // BR
    return pl.pallas_call(
        functools.partial(_pool_body, G=G, C=C, nsteps=nsteps),
        grid=(nsteps,),
        in_specs=[
            pl.BlockSpec((G, BR, C), lambda i: (0, i, 0)),
            pl.BlockSpec((1, 1, BR), lambda i: (i, 0, 0)),
        ],
        out_specs=pl.BlockSpec((G, NG, C), lambda i: (0, 0, 0)),
        out_shape=jax.ShapeDtypeStruct((G, NG, C), jnp.float32),
        scratch_shapes=[pltpu.VMEM((G, NG, C), jnp.float32),
                        pltpu.VMEM((NG, 1), jnp.float32)],
        interpret=_IP,
    )(h, batch3)


def _cls_body(h_ref, w_ref, b_ref, o_ref):
    for e in range(NE):
        o_ref[e] = jnp.dot(h_ref[e], w_ref[e],
                           preferred_element_type=jnp.float32) + b_ref[e]


def _cls(h_st, clsW, clsb):
    return pl.pallas_call(
        _cls_body,
        grid=(1,),
        in_specs=[
            pl.BlockSpec((NE, NG, H), lambda i: (0, 0, 0)),
            pl.BlockSpec((NE, H, NC), lambda i: (0, 0, 0)),
            pl.BlockSpec((NE, 1, NC), lambda i: (0, 0, 0)),
        ],
        out_specs=pl.BlockSpec((NE, NG, NC), lambda i: (0, 0, 0)),
        out_shape=jax.ShapeDtypeStruct((NE, NG, NC), jnp.float32),
        interpret=_IP,
    )(h_st, clsW, clsb.reshape(NE, 1, NC))


_NCORES = 2
_NSUB = 16
_NW = _NCORES * _NSUB
_EPT = E // _NW
_KE = 80
_NCH = _EPT // _KE
_RPT = N // _NSUB
_ZR = 25
_NCH16 = _KE // 16


def _sc_mesh():
    return plsc.VectorSubcoreMesh(core_axis_name="c", subcore_axis_name="s",
                                  num_cores=_NCORES, num_subcores=_NSUB)


_NB = 3
_NQ = (_NCH - 2) // _NB


def _sc_agg(h, pk, w, nexp, weighted):
    R = nexp * N
    NA = N + 8
    scratch = [
        pltpu.VMEM((_NB, 2, _KE), jnp.int32),
        pltpu.VMEM((_NB, _KE), jnp.float32),
        pltpu.VMEM((_NB, _KE, H), jnp.float32),
        pltpu.VMEM((_ZR, H), jnp.float32),
        pltpu.VMEM((_KE,), jnp.int32),
        pltpu.VMEM((_KE,), jnp.int32),
        pltpu.VMEM((_KE,), jnp.int32),
        pltpu.VMEM((_KE,), jnp.int32),
        pltpu.VMEM_SHARED((NA, H), jnp.float32),
    ] + [pltpu.SemaphoreType.DMA] * (2 * _NB)

    def body(h_hbm, pk_hbm, w_hbm, out_hbm,
             pkb, wvb, rows, zbuf, ddum, di0, di1, di2, acc, *sems):
        didx = (di0, di1, di2)
        gsem = sems[:_NB]
        ssem = sems[_NB:]
        c = lax.axis_index("c")
        s = lax.axis_index("s")
        wid = s * _NCORES + c
        cbase = wid * _NCH

        @pl.loop(0, _ZR)
        def _zb(i):
            for j in range(H // 16):
                zbuf[i, pl.ds(16 * j, 16)] = jnp.zeros((16,), jnp.float32)
        for j in range(_KE // 16):
            ddum[pl.ds(16 * j, 16)] = jnp.full((16,), N, jnp.int32)
        for b in range(_NB):
            pltpu.async_copy(rows.at[b], acc.at[ddum], ssem[b], add=True)

        def start(e, i, b):
            pltpu.make_async_copy(
                h_hbm.at[pl.ds(0, _KE), :], rows.at[b], ssem[b]).wait()
            pltpu.sync_copy(pk_hbm.at[cbase + i], pkb.at[b])
            if e > 0:
                for j in range(_KE // 16):
                    pkb[b, 0, pl.ds(16 * j, 16)] = (
                        pkb[b, 0, pl.ds(16 * j, 16)] + e * N)
            if weighted:
                pltpu.sync_copy(w_hbm.at[e, cbase + i], wvb.at[b])
            pltpu.async_copy(h_hbm.at[pkb.at[b, 0]], rows.at[b], gsem[b])

        def finish(b):
            pltpu.make_async_copy(
                h_hbm.at[pl.ds(0, _KE), :], rows.at[b], gsem[b]).wait()
            if weighted:
                @pl.loop(0, _NCH16)
                def _mul(k16):
                    wvec = wvb[b, pl.ds(k16 * 16, 16)]
                    for k2 in range(16):
                        wb = jnp.full((16,), wvec[k2], jnp.float32)
                        for j in range(H // 16):
                            rows[b, k16 * 16 + k2, pl.ds(16 * j, 16)] = (
                                rows[b, k16 * 16 + k2, pl.ds(16 * j, 16)]
                                * wb)
            for j in range(_KE // 16):
                didx[b][pl.ds(16 * j, 16)] = pkb[b, 1, pl.ds(16 * j, 16)]
            pltpu.async_copy(rows.at[b], acc.at[didx[b]], ssem[b],
                             add=True)

        for e in range(nexp):
            for z in range(_RPT // _ZR):
                pltpu.sync_copy(zbuf, acc.at[pl.ds(s * _RPT + z * _ZR, _ZR), :])
            plsc.subcore_barrier()

            start(e, 0, 0)
            start(e, 1, 1)

            @pl.loop(0, _NQ)
            def _ring(q):
                i = q * _NB
                start(e, i + 2, 2)
                finish(0)
                start(e, i + 3, 0)
                finish(1)
                start(e, i + 4, 1)
                finish(2)

            finish(0)
            finish(1)
            for b in range(_NB):
                pltpu.make_async_copy(
                    h_hbm.at[pl.ds(0, _KE), :], rows.at[b], ssem[b]).wait()
                pltpu.async_copy(rows.at[b], acc.at[ddum], ssem[b], add=True)
            plsc.subcore_barrier()
            pltpu.sync_copy(
                acc.at[pl.ds(s * _RPT, _RPT), :],
                out_hbm.at[c, pl.ds(e * N + s * _RPT, _RPT), :])
            plsc.subcore_barrier()
        for b in range(_NB):
            pltpu.make_async_copy(
                h_hbm.at[pl.ds(0, _KE), :], rows.at[b], ssem[b]).wait()

    if w is None:
        w = jnp.zeros((nexp, E // _KE, _KE), jnp.float32)
    return pl.kernel(
        body,
        out_type=jax.ShapeDtypeStruct((_NCORES, R, H), jnp.float32),
        mesh=_sc_mesh(),
        scratch_types=scratch,
        compiler_params=pltpu.CompilerParams(use_tc_tiling_on_sc=False),
    )(h, pk, w)


def _sc_edge_gather(Z, pk):
    scratch = [
        pltpu.VMEM((_NB, 2, _KE), jnp.int32),
        pltpu.VMEM((_NB, _KE, H), jnp.float32),
    ] + [pltpu.SemaphoreType.DMA] * (2 * _NB)

    def body(z_hbm, pk_hbm, out_hbm, pkb, rows, *sems):
        gsem = sems[:_NB]
        wsem = sems[_NB:]
        c = lax.axis_index("c")
        s = lax.axis_index("s")
        wid = s * _NCORES + c
        cbase = wid * _NCH
        eoff = wid * _EPT
        for b in range(_NB):
            pltpu.async_copy(rows.at[b],
                             out_hbm.at[0, pl.ds(eoff + b * _KE, _KE), :],
                             wsem[b])

        for which in range(2):
            def start(i, b):
                pltpu.make_async_copy(
                    z_hbm.at[pl.ds(0, _KE), :], rows.at[b], wsem[b]).wait()
                pltpu.sync_copy(pk_hbm.at[cbase + i], pkb.at[b])
                pltpu.async_copy(z_hbm.at[pkb.at[b, which]], rows.at[b],
                                 gsem[b])

            def finish(i, b):
                pltpu.make_async_copy(
                    z_hbm.at[pl.ds(0, _KE), :], rows.at[b], gsem[b]).wait()
                pltpu.async_copy(
                    rows.at[b],
                    out_hbm.at[which, pl.ds(eoff + i * _KE, _KE), :],
                    wsem[b])

            start(0, 0)
            start(1, 1)

            @pl.loop(0, _NQ)
            def _ring(q):
                i = q * _NB
                start(i + 2, 2)
                finish(i, 0)
                start(i + 3, 0)
                finish(i + 1, 1)
                start(i + 4, 1)
                finish(i + 2, 2)

            i0 = _NQ * _NB
            finish(i0, 0)
            finish(i0 + 1, 1)
        for b in range(_NB):
            pltpu.make_async_copy(
                z_hbm.at[pl.ds(0, _KE), :], rows.at[b], wsem[b]).wait()

    return pl.kernel(
        body,
        out_type=jax.ShapeDtypeStruct((2, E, H), jnp.float32),
        mesh=_sc_mesh(),
        scratch_types=scratch,
        compiler_params=pltpu.CompilerParams(use_tc_tiling_on_sc=False),
    )(Z, pk)


def _agg_jax(h, src, dst, w, nexp):
    hh = h.reshape(nexp, N, H)
    if w is None:
        w = jnp.ones((nexp, E), jnp.float32)
    def one(he, we):
        msg = he[src] * we[:, None]
        return jax.ops.segment_sum(msg, dst, num_segments=N)
    agg = jax.vmap(one)(hh, w)
    return agg.reshape(1, nexp * N, H)


def kernel(x, edge_index, batch, enc1_W1, enc1_b1, enc1_W2, enc1_b2,
           enc1_eps, enc2_W1, enc2_b1, enc2_W2, enc2_b2, enc2_eps,
           nmW1, nmb1, nmW2, nmb2, emW1, emb1, emW2, emb2,
           fmW1, fmb1, fmW2, fmb2, clsW, clsb):
    pk = jnp.stack([edge_index[0].reshape(E // _KE, _KE),
                    edge_index[1].reshape(E // _KE, _KE)], axis=1)
    batch3 = batch.reshape(N // 1000, 1, 1000)

    h = x
    for l in range(L):
        agg = _sc_agg(h, pk, None, 1, False)
        h = _gin_mlp(h, agg, enc1_W1[l], enc1_b1[l], enc1_W2[l],
                     enc1_b2[l], enc1_eps[l], relu_out=(l < L - 1))
    Z = h

    nmask, fmaskT, mx = _mask_stage(
        Z, x, nmW1, nmb1, nmW2, nmb2, fmW1, fmb1, fmW2, fmb2)

    ZsZd = _sc_edge_gather(Z, pk)
    emask = _edge_mlp(ZsZd[0], ZsZd[1], emW1, emb1, emW2, emb2)
    emaskT = emask.T.reshape(NE, E // _KE, _KE)

    h = mx.reshape(NE * N, H)
    for l in range(L):
        agg = _sc_agg(h, pk, emaskT, NE, True)
        h = _gin_mlp(h, agg, enc2_W1[l], enc2_b1[l], enc2_W2[l],
                     enc2_b2[l], enc2_eps[l], relu_out=(l < L - 1))
    mZ = h.reshape(NE, N, H)

    h_orig = _pool(Z.reshape(1, N, H), batch3)[0]
    h_st = _pool(mZ, batch3)
    logits = _cls(h_st, clsW, clsb)

    expert_logits = logits.transpose(1, 0, 2)
    h_stable = h_st.transpose(1, 0, 2)
    node_masks = nmask[:, :, None]
    edge_masks = emask[:, :, None]
    feat_masks = fmaskT.transpose(1, 0, 2)
    return (expert_logits, h_stable, h_orig, node_masks, edge_masks,
            feat_masks)
